# baseline (speedup 1.0000x reference)
import jax
import jax.numpy as jnp
from jax import lax
from jax.experimental import pallas as pl
from jax.experimental.pallas import tpu as pltpu
from jax.experimental.pallas import tpu_sc as plsc

_L, _N, _Q, _C = 6, 32, 900, 256
_NT = 16
_NSC = 8
_NSC_LOG2 = 3
_B = 8
_CH = 128
_NCH = _Q // _CH
_TAIL = _Q - _NCH * _CH
_NL = 16
_NJ = _C // _NL
_U = 2


def _tc_body(x_ref, out_ref):
    for b in range(_B):
        x = x_ref[0, b]
        q = x.shape[0]
        s = jnp.sum(x, axis=0, keepdims=True)
        c = s * (1.0 / q)
        n = jnp.sum(x * x, axis=1, keepdims=True)
        xc = jnp.dot(x, c.T, preferred_element_type=jnp.float32)
        score = n - 2.0 * xc
        idx = jnp.argmax(score[:, 0], axis=0)
        out_ref[0, b, 0:1, :] = c
        out_ref[0, b, 1:2, :] = x_ref[0, b, pl.ds(idx, 1), :]


def _tc_call(hs_pair):
    return pl.pallas_call(
        _tc_body,
        grid=(_L, _NT // _B),
        in_specs=[pl.BlockSpec((1, _B, _Q, _C), lambda i, j: (i, j, 0, 0))],
        out_specs=pl.BlockSpec((1, _B, 2, _C), lambda i, j: (i, j, 0, 0)),
        out_shape=jax.ShapeDtypeStruct((_L, _NT, 2, _C), jnp.float32),
    )(hs_pair)


def _lanesum(v):
    iota = lax.iota(jnp.int32, _NL)
    dnums = lax.GatherDimensionNumbers(
        offset_dims=(), collapsed_slice_dims=(0,), start_index_map=(0,))
    for sh in (8, 4, 2, 1):
        perm = (iota ^ sh)[:, None]
        v = v + lax.gather(v, perm, dnums, (1,),
                           mode=lax.GatherScatterMode.PROMISE_IN_BOUNDS)
    return v[0]


def _make_sc_body(n_base):
    def _sc_body(hs, out, buf0, buf1, tailbuf, rowbuf, npart, stage, idxr,
                 sem0, sem1):
        wid = lax.axis_index("s")
        bufs = (buf0, buf1)
        sems = (sem0, sem1)
        spt = (_L * _NSC) // 16
        iota = lax.iota(jnp.int32, _NL)

        idxr[pl.ds(0, _NL)] = jnp.minimum(_NCH * _CH + iota, _Q - 1)

        def one_slice(t, _):
            s_idx = wid * spt + t
            l = lax.shift_right_logical(s_idx, _NSC_LOG2)
            nn = lax.bitwise_and(s_idx, _NSC - 1)
            n = nn + n_base
            sl = hs.at[l, n]

            def stream(process, process_tail):
                copies = [pltpu.async_copy(
                    sl.at[pl.ds(0, _CH)], bufs[0], sems[0])]
                tail_cp = pltpu.async_copy(
                    sl.at[idxr.at[pl.ds(0, _NL)]], tailbuf, sems[1])
                for k in range(_NCH):
                    copies[k].wait()
                    if k + 1 < _NCH:
                        copies.append(pltpu.async_copy(
                            sl.at[pl.ds((k + 1) * _CH, _CH)],
                            bufs[(k + 1) % 2], sems[(k + 1) % 2]))
                    process(k, bufs[k % 2])
                tail_cp.wait()
                process_tail()

            state = {"S": tuple(jnp.zeros((_NL,), jnp.float32)
                                for _ in range(_NJ))}

            def p1_row(buf, r, rg, S):
                xs = [buf[r, pl.ds(j * _NL, _NL)] for j in range(_NJ)]
                acc = xs[0] * xs[0]
                for j in range(1, _NJ):
                    acc = acc + xs[j] * xs[j]
                npart[pl.ds(rg * _NL, _NL)] = acc
                return tuple(S[j] + xs[j] for j in range(_NJ))

            def p1(k, buf):
                def body(r, S):
                    for u in range(_U):
                        S = p1_row(buf, r * _U + u, k * _CH + r * _U + u, S)
                    return S
                state["S"] = lax.fori_loop(0, _CH // _U, body, state["S"])

            def p1_tail():
                S = state["S"]
                for r in range(_TAIL):
                    S = p1_row(tailbuf, r, _NCH * _CH + r, S)
                state["S"] = S

            stream(p1, p1_tail)
            cs = [state["S"][j] * (1.0 / _Q) for j in range(_NJ)]
            for j in range(_NJ):
                stage[0, pl.ds(j * _NL, _NL)] = cs[j]

            state2 = {"b": (jnp.float32(-jnp.inf), jnp.int32(0))}

            def p2_row(buf, r, rg, carry):
                bs, bi = carry
                xs = [buf[r, pl.ds(j * _NL, _NL)] for j in range(_NJ)]
                dot = xs[0] * cs[0]
                for j in range(1, _NJ):
                    dot = dot + xs[j] * cs[j]
                sv = npart[pl.ds(rg * _NL, _NL)] - 2.0 * dot
                s_sc = _lanesum(sv)
                better = s_sc > bs
                return (lax.select(better, s_sc, bs),
                        lax.select(better, jnp.int32(0) + rg, bi))

            def p2(k, buf):
                def body(r, c):
                    for u in range(_U):
                        c = p2_row(buf, r * _U + u, k * _CH + r * _U + u, c)
                    return c
                state2["b"] = lax.fori_loop(0, _CH // _U, body, state2["b"])

            def p2_tail():
                c = state2["b"]
                for r in range(_TAIL):
                    c = p2_row(tailbuf, r, _NCH * _CH + r, c)
                state2["b"] = c

            stream(p2, p2_tail)
            _, bi = state2["b"]

            idxr[pl.ds(_NL, _NL)] = jnp.int32(0) * iota + bi
            pltpu.async_copy(sl.at[idxr.at[pl.ds(_NL, 8)]], rowbuf,
                             sems[0]).wait()
            for j in range(_NJ):
                stage[1, pl.ds(j * _NL, _NL)] = rowbuf[0,
                                                       pl.ds(j * _NL, _NL)]
            pltpu.sync_copy(stage, out.at[l, nn])
            return 0

        lax.fori_loop(0, spt, one_slice, 0)

    return _sc_body


def _make_sc_call(n_base):
    return pl.kernel(
        _make_sc_body(n_base),
        out_type=jax.ShapeDtypeStruct((_L, _NSC, 2, _C), jnp.float32),
        mesh=plsc.VectorSubcoreMesh(core_axis_name="c", subcore_axis_name="s",
                                    num_cores=1),
        scratch_types=[
            pltpu.VMEM((_CH, _C), jnp.float32),
            pltpu.VMEM((_CH, _C), jnp.float32),
            pltpu.VMEM((_NL, _C), jnp.float32),
            pltpu.VMEM((8, _C), jnp.float32),
            pltpu.VMEM((_Q * _NL,), jnp.float32),
            pltpu.VMEM((2, _C), jnp.float32),
            pltpu.VMEM((2 * _NL,), jnp.int32),
            pltpu.SemaphoreType.DMA,
            pltpu.SemaphoreType.DMA,
        ],
    )


_sc_call0 = _make_sc_call(_NT)
_sc_call1 = _make_sc_call(_NT + _NSC)


def kernel(hs_pair):
    tc_out = _tc_call(hs_pair)
    sc0 = _sc_call0(hs_pair)
    sc1 = _sc_call1(hs_pair)
    return jnp.concatenate([tc_out, sc0, sc1], axis=1)


# speedup vs baseline: 1.4193x; 1.4193x over previous
import jax
import jax.numpy as jnp
from jax import lax
from jax.experimental import pallas as pl
from jax.experimental.pallas import tpu as pltpu
from jax.experimental.pallas import tpu_sc as plsc

_L, _N, _Q, _C = 6, 32, 900, 256
_NT = 24
_NSC = 8
_NSC_LOG2 = 3
_B = 8
_CH = 128
_NCH = _Q // _CH
_TAIL = _Q - _NCH * _CH
_NL = 16
_NJ = _C // _NL
_U = 2


def _tc_body(x_ref, out_ref):
    for b in range(_B):
        x = x_ref[0, b]
        q = x.shape[0]
        s = jnp.sum(x, axis=0, keepdims=True)
        c = s * (1.0 / q)
        n = jnp.sum(x * x, axis=1, keepdims=True)
        xc = jnp.dot(x, c.T, preferred_element_type=jnp.float32)
        score = n - 2.0 * xc
        idx = jnp.argmax(score[:, 0], axis=0)
        out_ref[0, b, 0:1, :] = c
        out_ref[0, b, 1:2, :] = x_ref[0, b, pl.ds(idx, 1), :]


def _tc_call(hs_pair):
    return pl.pallas_call(
        _tc_body,
        grid=(_L, _NT // _B),
        in_specs=[pl.BlockSpec((1, _B, _Q, _C), lambda i, j: (i, j, 0, 0))],
        out_specs=pl.BlockSpec((1, _B, 2, _C), lambda i, j: (i, j, 0, 0)),
        out_shape=jax.ShapeDtypeStruct((_L, _NT, 2, _C), jnp.float32),
    )(hs_pair)


def _lanesum(v):
    iota = lax.iota(jnp.int32, _NL)
    dnums = lax.GatherDimensionNumbers(
        offset_dims=(), collapsed_slice_dims=(0,), start_index_map=(0,))
    for sh in (8, 4, 2, 1):
        perm = (iota ^ sh)[:, None]
        v = v + lax.gather(v, perm, dnums, (1,),
                           mode=lax.GatherScatterMode.PROMISE_IN_BOUNDS)
    return v[0]


def _make_sc_body(n_base):
    def _sc_body(hs, out, buf0, buf1, tailbuf, rowbuf, npart, stage, idxr,
                 sem0, sem1):
        wid = lax.axis_index("s")
        bufs = (buf0, buf1)
        sems = (sem0, sem1)
        spt = (_L * _NSC) // 16
        iota = lax.iota(jnp.int32, _NL)

        idxr[pl.ds(0, _NL)] = jnp.minimum(_NCH * _CH + iota, _Q - 1)

        def one_slice(t, _):
            s_idx = wid * spt + t
            l = lax.shift_right_logical(s_idx, _NSC_LOG2)
            nn = lax.bitwise_and(s_idx, _NSC - 1)
            n = nn + n_base
            sl = hs.at[l, n]

            def stream(process, process_tail):
                copies = [pltpu.async_copy(
                    sl.at[pl.ds(0, _CH)], bufs[0], sems[0])]
                tail_cp = pltpu.async_copy(
                    sl.at[idxr.at[pl.ds(0, _NL)]], tailbuf, sems[1])
                for k in range(_NCH):
                    copies[k].wait()
                    if k + 1 < _NCH:
                        copies.append(pltpu.async_copy(
                            sl.at[pl.ds((k + 1) * _CH, _CH)],
                            bufs[(k + 1) % 2], sems[(k + 1) % 2]))
                    process(k, bufs[k % 2])
                tail_cp.wait()
                process_tail()

            state = {"S": tuple(jnp.zeros((_NL,), jnp.float32)
                                for _ in range(_NJ))}

            def p1_row(buf, r, rg, S):
                xs = [buf[r, pl.ds(j * _NL, _NL)] for j in range(_NJ)]
                acc = xs[0] * xs[0]
                for j in range(1, _NJ):
                    acc = acc + xs[j] * xs[j]
                npart[pl.ds(rg * _NL, _NL)] = acc
                return tuple(S[j] + xs[j] for j in range(_NJ))

            def p1(k, buf):
                def body(r, S):
                    for u in range(_U):
                        S = p1_row(buf, r * _U + u, k * _CH + r * _U + u, S)
                    return S
                state["S"] = lax.fori_loop(0, _CH // _U, body, state["S"])

            def p1_tail():
                S = state["S"]
                for r in range(_TAIL):
                    S = p1_row(tailbuf, r, _NCH * _CH + r, S)
                state["S"] = S

            stream(p1, p1_tail)
            cs = [state["S"][j] * (1.0 / _Q) for j in range(_NJ)]
            for j in range(_NJ):
                stage[0, pl.ds(j * _NL, _NL)] = cs[j]

            state2 = {"b": (jnp.float32(-jnp.inf), jnp.int32(0))}

            def p2_row(buf, r, rg, carry):
                bs, bi = carry
                xs = [buf[r, pl.ds(j * _NL, _NL)] for j in range(_NJ)]
                dot = xs[0] * cs[0]
                for j in range(1, _NJ):
                    dot = dot + xs[j] * cs[j]
                sv = npart[pl.ds(rg * _NL, _NL)] - 2.0 * dot
                s_sc = _lanesum(sv)
                better = s_sc > bs
                return (lax.select(better, s_sc, bs),
                        lax.select(better, jnp.int32(0) + rg, bi))

            def p2(k, buf):
                def body(r, c):
                    for u in range(_U):
                        c = p2_row(buf, r * _U + u, k * _CH + r * _U + u, c)
                    return c
                state2["b"] = lax.fori_loop(0, _CH // _U, body, state2["b"])

            def p2_tail():
                c = state2["b"]
                for r in range(_TAIL):
                    c = p2_row(tailbuf, r, _NCH * _CH + r, c)
                state2["b"] = c

            stream(p2, p2_tail)
            _, bi = state2["b"]

            idxr[pl.ds(_NL, _NL)] = jnp.int32(0) * iota + bi
            pltpu.async_copy(sl.at[idxr.at[pl.ds(_NL, 8)]], rowbuf,
                             sems[0]).wait()
            for j in range(_NJ):
                stage[1, pl.ds(j * _NL, _NL)] = rowbuf[0,
                                                       pl.ds(j * _NL, _NL)]
            pltpu.sync_copy(stage, out.at[l, nn])
            return 0

        lax.fori_loop(0, spt, one_slice, 0)

    return _sc_body


def _make_sc_call(n_base):
    return pl.kernel(
        _make_sc_body(n_base),
        out_type=jax.ShapeDtypeStruct((_L, _NSC, 2, _C), jnp.float32),
        mesh=plsc.VectorSubcoreMesh(core_axis_name="c", subcore_axis_name="s",
                                    num_cores=1),
        scratch_types=[
            pltpu.VMEM((_CH, _C), jnp.float32),
            pltpu.VMEM((_CH, _C), jnp.float32),
            pltpu.VMEM((_NL, _C), jnp.float32),
            pltpu.VMEM((8, _C), jnp.float32),
            pltpu.VMEM((_Q * _NL,), jnp.float32),
            pltpu.VMEM((2, _C), jnp.float32),
            pltpu.VMEM((2 * _NL,), jnp.int32),
            pltpu.SemaphoreType.DMA,
            pltpu.SemaphoreType.DMA,
        ],
    )


_sc_call0 = _make_sc_call(_NT)


def kernel(hs_pair):
    sc0 = _sc_call0(hs_pair)
    tc_out = _tc_call(hs_pair)
    return jnp.concatenate([tc_out, sc0], axis=1)


# speedup vs baseline: 1.4209x; 1.0012x over previous
import jax
import jax.numpy as jnp
from jax import lax
from jax.experimental import pallas as pl
from jax.experimental.pallas import tpu as pltpu
from jax.experimental.pallas import tpu_sc as plsc

_L, _N, _Q, _C = 6, 32, 900, 256
_NT = 24
_NSC = 8
_NSC_LOG2 = 3
_B = 8
_CH = 128
_NCH = _Q // _CH
_TAIL = _Q - _NCH * _CH
_NL = 16
_NJ = _C // _NL
_U = 2


def _tc_body(x_ref, out_ref):
    for b in range(_B):
        x = x_ref[0, b]
        q = x.shape[0]
        s = jnp.sum(x, axis=0, keepdims=True)
        c = s * (1.0 / q)
        n = jnp.sum(x * x, axis=1, keepdims=True)
        xc = jnp.dot(x, c.T, preferred_element_type=jnp.float32)
        score = n - 2.0 * xc
        idx = jnp.argmax(score[:, 0], axis=0)
        out_ref[0, b, 0:1, :] = c
        out_ref[0, b, 1:2, :] = x_ref[0, b, pl.ds(idx, 1), :]


def _tc_call(hs_pair):
    return pl.pallas_call(
        _tc_body,
        grid=(_L, _NT // _B),
        in_specs=[pl.BlockSpec((1, _B, _Q, _C), lambda i, j: (i, j, 0, 0))],
        out_specs=pl.BlockSpec((1, _B, 2, _C), lambda i, j: (i, j, 0, 0)),
        out_shape=jax.ShapeDtypeStruct((_L, _NT, 2, _C), jnp.float32),
    )(hs_pair)


def _lanesum(v):
    iota = lax.iota(jnp.int32, _NL)
    dnums = lax.GatherDimensionNumbers(
        offset_dims=(), collapsed_slice_dims=(0,), start_index_map=(0,))
    for sh in (8, 4, 2, 1):
        perm = (iota ^ sh)[:, None]
        v = v + lax.gather(v, perm, dnums, (1,),
                           mode=lax.GatherScatterMode.PROMISE_IN_BOUNDS)
    return v[0]


def _make_sc_body(n_base):
    def _sc_body(hs, out, buf0, buf1, tailbuf, rowbuf, npart, stage, idxr,
                 sem0, sem1):
        wid = lax.axis_index("s")
        bufs = (buf0, buf1)
        sems = (sem0, sem1)
        spt = (_L * _NSC) // 16
        iota = lax.iota(jnp.int32, _NL)

        idxr[pl.ds(0, _NL)] = jnp.minimum(_NCH * _CH + iota, _Q - 1)

        def one_slice(t, _):
            s_idx = wid * spt + t
            l = lax.shift_right_logical(s_idx, _NSC_LOG2)
            nn = lax.bitwise_and(s_idx, _NSC - 1)
            n = nn + n_base
            sl = hs.at[l, n]

            def stream(process, process_tail):
                copies = [pltpu.async_copy(
                    sl.at[pl.ds(0, _CH)], bufs[0], sems[0])]
                tail_cp = pltpu.async_copy(
                    sl.at[idxr.at[pl.ds(0, _NL)]], tailbuf, sems[1])
                for k in range(_NCH):
                    copies[k].wait()
                    if k + 1 < _NCH:
                        copies.append(pltpu.async_copy(
                            sl.at[pl.ds((k + 1) * _CH, _CH)],
                            bufs[(k + 1) % 2], sems[(k + 1) % 2]))
                    process(k, bufs[k % 2])
                tail_cp.wait()
                process_tail()

            state = {"S": tuple(jnp.zeros((_NL,), jnp.float32)
                                for _ in range(_NJ))}

            def p1_row(buf, r, rg, S):
                xs = [buf[r, pl.ds(j * _NL, _NL)] for j in range(_NJ)]
                acc = xs[0] * xs[0]
                for j in range(1, _NJ):
                    acc = acc + xs[j] * xs[j]
                npart[pl.ds(rg * _NL, _NL)] = acc
                return tuple(S[j] + xs[j] for j in range(_NJ))

            def p1(k, buf):
                def body(r, S):
                    for u in range(_U):
                        S = p1_row(buf, r * _U + u, k * _CH + r * _U + u, S)
                    return S
                state["S"] = lax.fori_loop(0, _CH // _U, body, state["S"])

            def p1_tail():
                S = state["S"]
                for r in range(_TAIL):
                    S = p1_row(tailbuf, r, _NCH * _CH + r, S)
                state["S"] = S

            stream(p1, p1_tail)
            cs = [state["S"][j] * (1.0 / _Q) for j in range(_NJ)]
            for j in range(_NJ):
                stage[0, pl.ds(j * _NL, _NL)] = cs[j]

            state2 = {"b": (jnp.float32(-jnp.inf), jnp.int32(0))}

            def p2_row(buf, r, rg, carry):
                bs, bi = carry
                xs = [buf[r, pl.ds(j * _NL, _NL)] for j in range(_NJ)]
                dot = xs[0] * cs[0]
                for j in range(1, _NJ):
                    dot = dot + xs[j] * cs[j]
                sv = npart[pl.ds(rg * _NL, _NL)] - 2.0 * dot
                s_sc = _lanesum(sv)
                better = s_sc > bs
                return (lax.select(better, s_sc, bs),
                        lax.select(better, jnp.int32(0) + rg, bi))

            def p2(k, buf):
                def body(r, c):
                    for u in range(_U):
                        c = p2_row(buf, r * _U + u, k * _CH + r * _U + u, c)
                    return c
                state2["b"] = lax.fori_loop(0, _CH // _U, body, state2["b"])

            def p2_tail():
                c = state2["b"]
                for r in range(_TAIL):
                    c = p2_row(tailbuf, r, _NCH * _CH + r, c)
                state2["b"] = c

            stream(p2, p2_tail)
            _, bi = state2["b"]

            idxr[pl.ds(_NL, _NL)] = jnp.int32(0) * iota + bi
            pltpu.async_copy(sl.at[idxr.at[pl.ds(_NL, 8)]], rowbuf,
                             sems[0]).wait()
            for j in range(_NJ):
                stage[1, pl.ds(j * _NL, _NL)] = rowbuf[0,
                                                       pl.ds(j * _NL, _NL)]
            pltpu.sync_copy(stage, out.at[l, nn])
            return 0

        lax.fori_loop(0, spt, one_slice, 0)

    return _sc_body


def _make_sc_call(n_base):
    return pl.kernel(
        _make_sc_body(n_base),
        out_type=jax.ShapeDtypeStruct((_L, _NSC, 2, _C), jnp.float32),
        mesh=plsc.VectorSubcoreMesh(core_axis_name="c", subcore_axis_name="s",
                                    num_cores=1),
        scratch_types=[
            pltpu.VMEM((_CH, _C), jnp.float32),
            pltpu.VMEM((_CH, _C), jnp.float32),
            pltpu.VMEM((_NL, _C), jnp.float32),
            pltpu.VMEM((8, _C), jnp.float32),
            pltpu.VMEM((_Q * _NL,), jnp.float32),
            pltpu.VMEM((2, _C), jnp.float32),
            pltpu.VMEM((2 * _NL,), jnp.int32),
            pltpu.SemaphoreType.DMA,
            pltpu.SemaphoreType.DMA,
        ],
    )


_sc_call0 = _make_sc_call(_NT)


def kernel(hs_pair):
    tc_out = _tc_call(hs_pair)
    sc0 = _sc_call0(hs_pair)
    return jnp.concatenate([tc_out, sc0], axis=1)


# speedup vs baseline: 1.7724x; 1.2474x over previous
import jax
import jax.numpy as jnp
from jax.experimental import pallas as pl


_B = 8


def _center_kernel(x_ref, out_ref):
    for b in range(_B):
        x = x_ref[0, b]
        q = x.shape[0]
        s = jnp.sum(x, axis=0, keepdims=True)
        c = s * (1.0 / q)
        n = jnp.sum(x * x, axis=1, keepdims=True)
        xc = jnp.dot(x, c.T, preferred_element_type=jnp.float32)
        score = n - 2.0 * xc
        idx = jnp.argmax(score[:, 0], axis=0)
        out_ref[0, b, 0:1, :] = c
        out_ref[0, b, 1:2, :] = x_ref[0, b, pl.ds(idx, 1), :]


def kernel(hs_pair):
    L, N, Q, C = hs_pair.shape
    return pl.pallas_call(
        _center_kernel,
        grid=(L, N // _B),
        in_specs=[pl.BlockSpec((1, _B, Q, C), lambda i, j: (i, j, 0, 0))],
        out_specs=pl.BlockSpec((1, _B, 2, C), lambda i, j: (i, j, 0, 0)),
        out_shape=jax.ShapeDtypeStruct((L, N, 2, C), hs_pair.dtype),
    )(hs_pair)
